# baseline (speedup 1.0000x reference)
import functools

import jax
import jax.numpy as jnp
from jax import lax
from jax.experimental import pallas as pl
from jax.experimental.pallas import tpu as pltpu
from jax.experimental.pallas import tpu_sc as plsc

_VOCAB = 1000
_D_EMB = 128
_D_MODEL = 512

_NC = 2
_NS = 16
_NW = _NC * _NS


def _fuse_body(w_ref, fw_ref, b_ref, o_ref):
    o_ref[...] = (
        jnp.dot(w_ref[...], fw_ref[...], preferred_element_type=jnp.float32)
        + b_ref[...]
    )


def _fused_table(weights, fc_w, fc_b):
    return pl.pallas_call(
        _fuse_body,
        out_shape=jax.ShapeDtypeStruct((_VOCAB, _D_MODEL), jnp.float32),
    )(weights, fc_w, fc_b.reshape(1, _D_MODEL))


def _make_gather(bs, seq, d, ch=32, nbuf=5):
    n_tok = bs * seq
    per_w = n_tok // _NW
    n_chunks = per_w // ch
    chunks_per_plane = bs // ch
    assert n_tok % _NW == 0 and per_w % ch == 0 and n_chunks % nbuf == 0
    assert bs % ch == 0 and ch % 8 == 0

    mesh = plsc.VectorSubcoreMesh(core_axis_name="c", subcore_axis_name="s")

    @functools.partial(
        pl.kernel,
        mesh=mesh,
        out_type=jax.ShapeDtypeStruct((seq, bs, d), jnp.float32),
        scratch_types=[
            pltpu.VMEM((per_w,), jnp.int32),
            pltpu.VMEM((nbuf, ch, d), jnp.float32),
        ]
        + [pltpu.SemaphoreType.DMA] * (2 * nbuf),
    )
    def gather(table_hbm, tok_hbm, out_hbm, idx_v, rows_v, *sems):
        gsem, wsem = sems[:nbuf], sems[nbuf:]
        wid = lax.axis_index("s") * _NC + lax.axis_index("c")
        gbase = wid * n_chunks
        pltpu.sync_copy(tok_hbm.at[pl.ds(wid * per_w, per_w)], idx_v)

        def g_copy(c, b):
            return pltpu.make_async_copy(
                table_hbm.at[idx_v.at[pl.ds(c * ch, ch)]], rows_v.at[b], gsem[b]
            )

        def w_copy(c, b):
            g = gbase + c
            sp = g // chunks_per_plane
            b0 = (g % chunks_per_plane) * ch
            return pltpu.make_async_copy(
                rows_v.at[b], out_hbm.at[sp, pl.ds(b0, ch)], wsem[b]
            )

        for c in range(3):
            g_copy(c, c % nbuf).start()

        def outer(i, carry):
            for j in range(nbuf):
                s = i * nbuf + j
                bg = (j + 3) % nbuf

                @pl.when((s + 3 < n_chunks) & (s - 2 >= 0))
                def _():
                    w_copy(0, bg).wait()

                @pl.when(s + 3 < n_chunks)
                def _():
                    g_copy(s + 3, bg).start()

                g_copy(0, j).wait()
                w_copy(s, j).start()
            return carry

        lax.fori_loop(0, n_chunks // nbuf, outer, 0)
        for j in range(nbuf):
            w_copy(0, j).wait()

    def run(table, tokens):
        toks = tokens.T.reshape(-1)
        out = gather(table, toks)
        return out.transpose(1, 0, 2)

    return run


def kernel(tokens, weights, fc_w, fc_b):
    bs, seq = tokens.shape
    fused = _fused_table(weights, fc_w, fc_b)
    return _make_gather(bs, seq, _D_MODEL)(fused, tokens.astype(jnp.int32))


# speedup vs baseline: 1.0039x; 1.0039x over previous
import functools

import jax
import jax.numpy as jnp
from jax import lax
from jax.experimental import pallas as pl
from jax.experimental.pallas import tpu as pltpu
from jax.experimental.pallas import tpu_sc as plsc

_VOCAB = 1000
_D_EMB = 128
_D_MODEL = 512

_NC = 2
_NS = 16
_NW = _NC * _NS


def _fuse_body(w_ref, fw_ref, b_ref, o_ref):
    o_ref[...] = (
        jnp.dot(w_ref[...], fw_ref[...], preferred_element_type=jnp.float32)
        + b_ref[...]
    )


def _fused_table(weights, fc_w, fc_b):
    return pl.pallas_call(
        _fuse_body,
        out_shape=jax.ShapeDtypeStruct((_VOCAB, _D_MODEL), jnp.float32),
    )(weights, fc_w, fc_b.reshape(1, _D_MODEL))


def _make_gather(bs, seq, d, ch=32, nbuf=5):
    n_tok = bs * seq
    per_w = n_tok // _NW
    n_chunks = per_w // ch
    chunks_per_plane = bs // ch
    assert n_tok % _NW == 0 and per_w % ch == 0 and n_chunks % nbuf == 0
    assert bs % ch == 0 and ch % 8 == 0

    mesh = plsc.VectorSubcoreMesh(core_axis_name="c", subcore_axis_name="s")

    @functools.partial(
        pl.kernel,
        mesh=mesh,
        out_type=jax.ShapeDtypeStruct((seq, bs, d), jnp.float32),
        scratch_types=[
            pltpu.VMEM((per_w,), jnp.int32),
            pltpu.VMEM((nbuf, ch, d), jnp.float32),
        ]
        + [pltpu.SemaphoreType.DMA] * (2 * nbuf),
    )
    def gather(table_hbm, tok_hbm, out_hbm, idx_v, rows_v, *sems):
        gsem, wsem = sems[:nbuf], sems[nbuf:]
        wid = lax.axis_index("s") * _NC + lax.axis_index("c")
        gbase = wid * n_chunks
        pltpu.sync_copy(tok_hbm.at[pl.ds(wid * per_w, per_w)], idx_v)

        def g_copy(c, b):
            return pltpu.make_async_copy(
                table_hbm.at[idx_v.at[pl.ds(c * ch, ch)]], rows_v.at[b], gsem[b]
            )

        def w_copy(c, b):
            g = gbase + c
            sp = g // chunks_per_plane
            b0 = (g % chunks_per_plane) * ch
            return pltpu.make_async_copy(
                rows_v.at[b], out_hbm.at[sp, pl.ds(b0, ch)], wsem[b]
            )

        for c in range(2):
            g_copy(c, c % nbuf).start()

        def outer(i, carry):
            for j in range(nbuf):
                s = i * nbuf + j
                bg = (j + 2) % nbuf

                @pl.when((s + 2 < n_chunks) & (s - 3 >= 0))
                def _():
                    w_copy(0, bg).wait()

                @pl.when(s + 2 < n_chunks)
                def _():
                    g_copy(s + 2, bg).start()

                g_copy(0, j).wait()
                w_copy(s, j).start()
            return carry

        lax.fori_loop(0, n_chunks // nbuf, outer, 0)
        for j in range(nbuf):
            w_copy(0, j).wait()

    def run(table, tokens):
        toks = tokens.T.reshape(-1)
        out = gather(table, toks)
        return out.transpose(1, 0, 2)

    return run


def kernel(tokens, weights, fc_w, fc_b):
    bs, seq = tokens.shape
    fused = _fused_table(weights, fc_w, fc_b)
    return _make_gather(bs, seq, _D_MODEL)(fused, tokens.astype(jnp.int32))


# speedup vs baseline: 1.0044x; 1.0005x over previous
import functools

import jax
import jax.numpy as jnp
from jax import lax
from jax.experimental import pallas as pl
from jax.experimental.pallas import tpu as pltpu
from jax.experimental.pallas import tpu_sc as plsc

_VOCAB = 1000
_D_EMB = 128
_D_MODEL = 512

_NC = 2
_NS = 16
_NW = _NC * _NS


def _fuse_body(w_ref, fw_ref, b_ref, o_ref):
    o_ref[...] = (
        jnp.dot(w_ref[...], fw_ref[...], preferred_element_type=jnp.float32)
        + b_ref[...]
    )


def _fused_table(weights, fc_w, fc_b):
    return pl.pallas_call(
        _fuse_body,
        out_shape=jax.ShapeDtypeStruct((_VOCAB, _D_MODEL), jnp.float32),
    )(weights, fc_w, fc_b.reshape(1, _D_MODEL))


def _make_gather(bs, seq, d, ch=64, nbuf=3):
    n_tok = bs * seq
    per_w = n_tok // _NW
    n_chunks = per_w // ch
    n_steady = (n_chunks - 1) // nbuf * nbuf
    chunks_per_plane = bs // ch
    assert n_tok % _NW == 0 and per_w % ch == 0
    assert bs % ch == 0 and ch % 8 == 0

    mesh = plsc.VectorSubcoreMesh(core_axis_name="c", subcore_axis_name="s")

    @functools.partial(
        pl.kernel,
        mesh=mesh,
        out_type=jax.ShapeDtypeStruct((seq, bs, d), jnp.float32),
        scratch_types=[
            pltpu.VMEM((per_w,), jnp.int32),
            pltpu.VMEM((nbuf, ch, d), jnp.float32),
        ]
        + [pltpu.SemaphoreType.DMA] * (2 * nbuf),
    )
    def gather(table_hbm, tok_hbm, out_hbm, idx_v, rows_v, *sems):
        gsem, wsem = sems[:nbuf], sems[nbuf:]
        wid = lax.axis_index("s") * _NC + lax.axis_index("c")
        gbase = wid * n_chunks
        pltpu.sync_copy(tok_hbm.at[pl.ds(wid * per_w, per_w)], idx_v)

        def g_copy(c, b):
            return pltpu.make_async_copy(
                table_hbm.at[idx_v.at[pl.ds(c * ch, ch)]], rows_v.at[b], gsem[b]
            )

        def w_copy(c, b):
            g = gbase + c
            sp = g // chunks_per_plane
            b0 = (g % chunks_per_plane) * ch
            return pltpu.make_async_copy(
                rows_v.at[b], out_hbm.at[sp, pl.ds(b0, ch)], wsem[b]
            )

        for c in range(2):
            g_copy(c, c % nbuf).start()

        def outer(i, carry):
            for j in range(nbuf):
                s = i * nbuf + j
                bg = (j + 2) % nbuf

                @pl.when((s + 2 < n_chunks) & (s + 2 - nbuf >= 0))
                def _():
                    w_copy(0, bg).wait()

                @pl.when(s + 2 < n_chunks)
                def _():
                    g_copy(s + 2, bg).start()

                g_copy(0, j).wait()
                w_copy(s, j).start()
            return carry

        lax.fori_loop(0, n_steady // nbuf, outer, 0)
        for s in range(n_steady, n_chunks):
            g_copy(0, s % nbuf).wait()
            w_copy(s, s % nbuf).start()
        for j in range(min(nbuf, n_chunks)):
            w_copy(0, j).wait()

    def run(table, tokens):
        toks = tokens.T.reshape(-1)
        out = gather(table, toks)
        return out.transpose(1, 0, 2)

    return run


def kernel(tokens, weights, fc_w, fc_b):
    bs, seq = tokens.shape
    fused = _fused_table(weights, fc_w, fc_b)
    return _make_gather(bs, seq, _D_MODEL)(fused, tokens.astype(jnp.int32))
